# baseline (speedup 1.0000x reference)
import functools

import jax
import jax.numpy as jnp
from jax import lax
from jax.experimental import pallas as pl
from jax.experimental.pallas import tpu as pltpu
from jax.experimental.pallas import tpu_sc as plsc

D = 64
DP = 128
B = 16384
NC = 2
NS = 16
NW = NC * NS
SIDE = B // NW
TOT = 2 * SIDE
CHUNK = 128
GROUPS = CHUNK // 16
NCHUNKS = TOT // CHUNK
NPAIR = NCHUNKS // 2


def _pad_body(x_ref, o_ref):
    blk = x_ref[...]
    o_ref[:, 0:D] = blk
    o_ref[:, D:DP] = jnp.zeros_like(blk)


def _pad128(x, blk_rows):
    n = x.shape[0]
    return pl.pallas_call(
        _pad_body,
        grid=(n // blk_rows,),
        in_specs=[pl.BlockSpec((blk_rows, D), lambda i: (i, 0))],
        out_specs=pl.BlockSpec((blk_rows, DP), lambda i: (i, 0)),
        out_shape=jax.ShapeDtypeStruct((n, DP), jnp.float32),
    )(x)


def _tec_body(pos_h, pos_t, pos_r, neg_h, neg_t, neg_r, ent, rel,
              pos_out, neg_out,
              hidx, tidx, ridx, h0, t0, r0, h1, t1, r1, outv, sem0, sem1):
    wid = lax.axis_index("s") * NC + lax.axis_index("c")
    base = wid * SIDE

    pltpu.sync_copy(pos_h.at[pl.ds(base, SIDE)], hidx.at[pl.ds(0, SIDE)])
    pltpu.sync_copy(neg_h.at[pl.ds(base, SIDE)], hidx.at[pl.ds(SIDE, SIDE)])
    pltpu.sync_copy(pos_t.at[pl.ds(base, SIDE)], tidx.at[pl.ds(0, SIDE)])
    pltpu.sync_copy(neg_t.at[pl.ds(base, SIDE)], tidx.at[pl.ds(SIDE, SIDE)])
    pltpu.sync_copy(pos_r.at[pl.ds(base, SIDE)], ridx.at[pl.ds(0, SIDE)])
    pltpu.sync_copy(neg_r.at[pl.ds(base, SIDE)], ridx.at[pl.ds(SIDE, SIDE)])

    def issue(off, hb, tb, rb, sem):
        pltpu.async_copy(ent.at[hidx.at[pl.ds(off, CHUNK)]], hb, sem)
        pltpu.async_copy(ent.at[tidx.at[pl.ds(off, CHUNK)]], tb, sem)
        pltpu.async_copy(rel.at[ridx.at[pl.ds(off, CHUNK)]], rb, sem)

    def drain(hb, tb, rb, sem):
        for buf in (hb, tb, rb):
            pltpu.make_async_copy(ent.at[pl.ds(0, CHUNK)], buf, sem).wait()

    lane = lax.iota(jnp.int32, 16)
    perms = [lane ^ (1 << b) for b in range(4)]
    dnums = lax.GatherDimensionNumbers(
        offset_dims=(), collapsed_slice_dims=(0,), start_index_map=(0,))

    def shuffle(v, perm):
        return lax.gather(
            v, perm[:, None], dimension_numbers=dnums, slice_sizes=(1,),
            mode=lax.GatherScatterMode.PROMISE_IN_BOUNDS)

    def compute(off, hb, tb, rb):
        def group_body(g, carry):
            acc = jnp.zeros((16,), jnp.float32)
            for l in range(16):
                row = g * 16 + l
                p = jnp.zeros((16,), jnp.float32)
                for k in range(D // 16):
                    hv = hb[row, pl.ds(k * 16, 16)]
                    tv = tb[row, pl.ds(k * 16, 16)]
                    rv = rb[row, pl.ds(k * 16, 16)]
                    p = p + jnp.abs(hv + rv - tv)
                for bstep in range(4):
                    p = p + shuffle(p, perms[bstep])
                acc = jnp.where(lane == l, p, acc)
            outv[pl.ds(off + g * 16, 16)] = acc
            return carry
        lax.fori_loop(0, GROUPS, group_body, 0)

    issue(0, h0, t0, r0, sem0)

    def pair_body(pidx, carry):
        off0 = pl.multiple_of(2 * pidx * CHUNK, CHUNK)
        off1 = pl.multiple_of(off0 + CHUNK, CHUNK)
        issue(off1, h1, t1, r1, sem1)
        drain(h0, t0, r0, sem0)
        compute(off0, h0, t0, r0)

        @pl.when(pidx < NPAIR - 1)
        def _():
            issue(off1 + CHUNK, h0, t0, r0, sem0)

        drain(h1, t1, r1, sem1)
        compute(off1, h1, t1, r1)
        return carry

    lax.fori_loop(0, NPAIR, pair_body, 0)

    pltpu.sync_copy(outv.at[pl.ds(0, SIDE)], pos_out.at[pl.ds(base, SIDE)])
    pltpu.sync_copy(outv.at[pl.ds(SIDE, SIDE)], neg_out.at[pl.ds(base, SIDE)])


@functools.partial(jax.jit, donate_argnums=())
def _run(pos_h, pos_t, pos_r, neg_h, neg_t, neg_r, ent_emb, rel_emb):
    ent_pad = _pad128(ent_emb, 4096)
    rel_pad = _pad128(rel_emb, 1000)
    mesh = plsc.VectorSubcoreMesh(core_axis_name="c", subcore_axis_name="s")
    k = pl.kernel(
        _tec_body,
        mesh=mesh,
        compiler_params=pltpu.CompilerParams(use_tc_tiling_on_sc=False),
        out_type=(
            jax.ShapeDtypeStruct((B,), jnp.float32),
            jax.ShapeDtypeStruct((B,), jnp.float32),
        ),
        scratch_types=[
            pltpu.VMEM((TOT,), jnp.int32),
            pltpu.VMEM((TOT,), jnp.int32),
            pltpu.VMEM((TOT,), jnp.int32),
            pltpu.VMEM((CHUNK, DP), jnp.float32),
            pltpu.VMEM((CHUNK, DP), jnp.float32),
            pltpu.VMEM((CHUNK, DP), jnp.float32),
            pltpu.VMEM((CHUNK, DP), jnp.float32),
            pltpu.VMEM((CHUNK, DP), jnp.float32),
            pltpu.VMEM((CHUNK, DP), jnp.float32),
            pltpu.VMEM((TOT,), jnp.float32),
            pltpu.SemaphoreType.DMA,
            pltpu.SemaphoreType.DMA,
        ],
    )
    return k(pos_h, pos_t, pos_r, neg_h, neg_t, neg_r, ent_pad, rel_pad)


def kernel(pos_h, pos_t, pos_r, neg_h, neg_t, neg_r, ent_emb, rel_emb):
    idx = [jnp.asarray(a, jnp.int32)
           for a in (pos_h, pos_t, pos_r, neg_h, neg_t, neg_r)]
    return _run(*idx, ent_emb, rel_emb)


# speedup vs baseline: 1.2775x; 1.2775x over previous
import functools

import jax
import jax.numpy as jnp
from jax import lax
from jax.experimental import pallas as pl
from jax.experimental.pallas import tpu as pltpu
from jax.experimental.pallas import tpu_sc as plsc

D = 64
DP = 128
B = 16384
NC = 2
NS = 16
NW = NC * NS
SIDE = B // NW
TOT = 2 * SIDE
CHUNK = 128
GROUPS = CHUNK // 16
NCHUNKS = TOT // CHUNK
NPAIR = NCHUNKS // 2


def _pad_body(x_ref, o_ref):
    blk = x_ref[...]
    o_ref[:, 0:D] = blk
    o_ref[:, D:DP] = jnp.zeros_like(blk)


def _pad128(x, blk_rows):
    n = x.shape[0]
    return pl.pallas_call(
        _pad_body,
        grid=(n // blk_rows,),
        in_specs=[pl.BlockSpec((blk_rows, D), lambda i: (i, 0))],
        out_specs=pl.BlockSpec((blk_rows, DP), lambda i: (i, 0)),
        out_shape=jax.ShapeDtypeStruct((n, DP), jnp.float32),
    )(x)


def _tec_body(pos_h, pos_t, pos_r, neg_h, neg_t, neg_r, ent, rel,
              pos_out, neg_out,
              hidx, tidx, ridx, h0, t0, r0, h1, t1, r1, outv, sem0, sem1):
    wid = lax.axis_index("s") * NC + lax.axis_index("c")
    base = wid * SIDE

    pltpu.sync_copy(pos_h.at[pl.ds(base, SIDE)], hidx.at[pl.ds(0, SIDE)])
    pltpu.sync_copy(neg_h.at[pl.ds(base, SIDE)], hidx.at[pl.ds(SIDE, SIDE)])
    pltpu.sync_copy(pos_t.at[pl.ds(base, SIDE)], tidx.at[pl.ds(0, SIDE)])
    pltpu.sync_copy(neg_t.at[pl.ds(base, SIDE)], tidx.at[pl.ds(SIDE, SIDE)])
    pltpu.sync_copy(pos_r.at[pl.ds(base, SIDE)], ridx.at[pl.ds(0, SIDE)])
    pltpu.sync_copy(neg_r.at[pl.ds(base, SIDE)], ridx.at[pl.ds(SIDE, SIDE)])

    def issue(off, hb, tb, rb, sem):
        pltpu.async_copy(ent.at[hidx.at[pl.ds(off, CHUNK)]], hb, sem)
        pltpu.async_copy(ent.at[tidx.at[pl.ds(off, CHUNK)]], tb, sem)
        pltpu.async_copy(rel.at[ridx.at[pl.ds(off, CHUNK)]], rb, sem)

    def drain(hb, tb, rb, sem):
        for buf in (hb, tb, rb):
            pltpu.make_async_copy(ent.at[pl.ds(0, CHUNK)], buf, sem).wait()

    lane = lax.iota(jnp.int32, 16)
    perms = [lane ^ (1 << b) for b in range(4)]
    dnums = lax.GatherDimensionNumbers(
        offset_dims=(), collapsed_slice_dims=(0,), start_index_map=(0,))

    def shuffle(v, perm):
        return lax.gather(
            v, perm[:, None], dimension_numbers=dnums, slice_sizes=(1,),
            mode=lax.GatherScatterMode.PROMISE_IN_BOUNDS)

    def compute(off, hb, tb, rb):
        def group_body(g, carry):
            acc = jnp.zeros((16,), jnp.float32)
            for l in range(16):
                row = g * 16 + l
                p = jnp.zeros((16,), jnp.float32)
                for k in range(D // 16):
                    hv = hb[row, pl.ds(k * 16, 16)]
                    tv = tb[row, pl.ds(k * 16, 16)]
                    rv = rb[row, pl.ds(k * 16, 16)]
                    p = p + jnp.abs(hv + rv - tv)
                for bstep in range(4):
                    p = p + shuffle(p, perms[bstep])
                acc = jnp.where(lane == l, p, acc)
            outv[pl.ds(off + g * 16, 16)] = acc
            return carry
        lax.fori_loop(0, GROUPS, group_body, 0)

    issue(0, h0, t0, r0, sem0)

    def pair_body(pidx, carry):
        off0 = pl.multiple_of(2 * pidx * CHUNK, CHUNK)
        off1 = pl.multiple_of(off0 + CHUNK, CHUNK)
        issue(off1, h1, t1, r1, sem1)
        drain(h0, t0, r0, sem0)
        compute(off0, h0, t0, r0)

        @pl.when(pidx < NPAIR - 1)
        def _():
            issue(off1 + CHUNK, h0, t0, r0, sem0)

        drain(h1, t1, r1, sem1)
        compute(off1, h1, t1, r1)
        return carry

    lax.fori_loop(0, NPAIR, pair_body, 0)

    pltpu.sync_copy(outv.at[pl.ds(0, SIDE)], pos_out.at[pl.ds(base, SIDE)])
    pltpu.sync_copy(outv.at[pl.ds(SIDE, SIDE)], neg_out.at[pl.ds(base, SIDE)])


@functools.partial(jax.jit, donate_argnums=())
def _run(pos_h, pos_t, pos_r, neg_h, neg_t, neg_r, ent_emb, rel_emb):
    ent_pad = jnp.pad(ent_emb, ((0, 0), (0, DP - D)))
    rel_pad = jnp.pad(rel_emb, ((0, 0), (0, DP - D)))
    mesh = plsc.VectorSubcoreMesh(core_axis_name="c", subcore_axis_name="s")
    k = pl.kernel(
        _tec_body,
        mesh=mesh,
        compiler_params=pltpu.CompilerParams(use_tc_tiling_on_sc=False),
        out_type=(
            jax.ShapeDtypeStruct((B,), jnp.float32),
            jax.ShapeDtypeStruct((B,), jnp.float32),
        ),
        scratch_types=[
            pltpu.VMEM((TOT,), jnp.int32),
            pltpu.VMEM((TOT,), jnp.int32),
            pltpu.VMEM((TOT,), jnp.int32),
            pltpu.VMEM((CHUNK, DP), jnp.float32),
            pltpu.VMEM((CHUNK, DP), jnp.float32),
            pltpu.VMEM((CHUNK, DP), jnp.float32),
            pltpu.VMEM((CHUNK, DP), jnp.float32),
            pltpu.VMEM((CHUNK, DP), jnp.float32),
            pltpu.VMEM((CHUNK, DP), jnp.float32),
            pltpu.VMEM((TOT,), jnp.float32),
            pltpu.SemaphoreType.DMA,
            pltpu.SemaphoreType.DMA,
        ],
    )
    return k(pos_h, pos_t, pos_r, neg_h, neg_t, neg_r, ent_pad, rel_pad)


def kernel(pos_h, pos_t, pos_r, neg_h, neg_t, neg_r, ent_emb, rel_emb):
    idx = [jnp.asarray(a, jnp.int32)
           for a in (pos_h, pos_t, pos_r, neg_h, neg_t, neg_r)]
    return _run(*idx, ent_emb, rel_emb)


# speedup vs baseline: 1.2791x; 1.0013x over previous
import functools

import jax
import jax.numpy as jnp
from jax import lax
from jax.experimental import pallas as pl
from jax.experimental.pallas import tpu as pltpu
from jax.experimental.pallas import tpu_sc as plsc

D = 64
DP = 128
B = 16384
NC = 2
NS = 16
NW = NC * NS
SIDE = B // NW
TOT = 2 * SIDE
CHUNK = 128
GROUPS = CHUNK // 16
NCHUNKS = TOT // CHUNK
NPAIR = NCHUNKS // 2


def _tec_body(pos_h, pos_t, pos_r, neg_h, neg_t, neg_r, ent, rel,
              pos_out, neg_out,
              hidx, tidx, ridx, h0, t0, r0, h1, t1, r1, outv, sem0, sem1):
    wid = lax.axis_index("s") * NC + lax.axis_index("c")
    base = wid * SIDE

    pltpu.sync_copy(pos_h.at[pl.ds(base, SIDE)], hidx.at[pl.ds(0, SIDE)])
    pltpu.sync_copy(neg_h.at[pl.ds(base, SIDE)], hidx.at[pl.ds(SIDE, SIDE)])
    pltpu.sync_copy(pos_t.at[pl.ds(base, SIDE)], tidx.at[pl.ds(0, SIDE)])
    pltpu.sync_copy(neg_t.at[pl.ds(base, SIDE)], tidx.at[pl.ds(SIDE, SIDE)])
    pltpu.sync_copy(pos_r.at[pl.ds(base, SIDE)], ridx.at[pl.ds(0, SIDE)])
    pltpu.sync_copy(neg_r.at[pl.ds(base, SIDE)], ridx.at[pl.ds(SIDE, SIDE)])

    def issue(off, hb, tb, rb, sem):
        pltpu.async_copy(ent.at[hidx.at[pl.ds(off, CHUNK)]], hb, sem)
        pltpu.async_copy(ent.at[tidx.at[pl.ds(off, CHUNK)]], tb, sem)
        pltpu.async_copy(rel.at[ridx.at[pl.ds(off, CHUNK)]], rb, sem)

    def drain(hb, tb, rb, sem):
        for buf in (hb, tb, rb):
            pltpu.make_async_copy(ent.at[pl.ds(0, CHUNK)], buf, sem).wait()

    lane = lax.iota(jnp.int32, 16)
    perms = [lane ^ (1 << b) for b in range(4)]
    dnums = lax.GatherDimensionNumbers(
        offset_dims=(), collapsed_slice_dims=(0,), start_index_map=(0,))

    def shuffle(v, perm):
        return lax.gather(
            v, perm[:, None], dimension_numbers=dnums, slice_sizes=(1,),
            mode=lax.GatherScatterMode.PROMISE_IN_BOUNDS)

    def compute(off, hb, tb, rb):
        def group_body(g, carry):
            acc = jnp.zeros((16,), jnp.float32)
            for l in range(16):
                row = g * 16 + l
                p = jnp.zeros((16,), jnp.float32)
                for k in range(D // 16):
                    hv = hb[row, pl.ds(k * 16, 16)]
                    tv = tb[row, pl.ds(k * 16, 16)]
                    rv = rb[row, pl.ds(k * 16, 16)]
                    p = p + jnp.abs(hv + rv - tv)
                for bstep in range(4):
                    p = p + shuffle(p, perms[bstep])
                acc = jnp.where(lane == l, p, acc)
            outv[pl.ds(off + g * 16, 16)] = acc
            return carry
        lax.fori_loop(0, GROUPS, group_body, 0)

    issue(0, h0, t0, r0, sem0)

    def pair_body(pidx, carry):
        off0 = pl.multiple_of(2 * pidx * CHUNK, CHUNK)
        off1 = pl.multiple_of(off0 + CHUNK, CHUNK)
        issue(off1, h1, t1, r1, sem1)
        drain(h0, t0, r0, sem0)
        compute(off0, h0, t0, r0)

        @pl.when(pidx < NPAIR - 1)
        def _():
            issue(off1 + CHUNK, h0, t0, r0, sem0)

        drain(h1, t1, r1, sem1)
        compute(off1, h1, t1, r1)
        return carry

    lax.fori_loop(0, NPAIR, pair_body, 0)

    pltpu.sync_copy(outv.at[pl.ds(0, SIDE)], pos_out.at[pl.ds(base, SIDE)])
    pltpu.sync_copy(outv.at[pl.ds(SIDE, SIDE)], neg_out.at[pl.ds(base, SIDE)])


@functools.partial(jax.jit, donate_argnums=())
def _run(pos_h, pos_t, pos_r, neg_h, neg_t, neg_r, ent_emb, rel_emb):
    ent_pad = jnp.pad(ent_emb, ((0, 0), (0, DP - D)))
    rel_pad = jnp.pad(rel_emb, ((0, 0), (0, DP - D)))
    mesh = plsc.VectorSubcoreMesh(core_axis_name="c", subcore_axis_name="s")
    k = pl.kernel(
        _tec_body,
        mesh=mesh,
        compiler_params=pltpu.CompilerParams(use_tc_tiling_on_sc=False),
        out_type=(
            jax.ShapeDtypeStruct((B,), jnp.float32),
            jax.ShapeDtypeStruct((B,), jnp.float32),
        ),
        scratch_types=[
            pltpu.VMEM((TOT,), jnp.int32),
            pltpu.VMEM((TOT,), jnp.int32),
            pltpu.VMEM((TOT,), jnp.int32),
            pltpu.VMEM((CHUNK, DP), jnp.float32),
            pltpu.VMEM((CHUNK, DP), jnp.float32),
            pltpu.VMEM((CHUNK, DP), jnp.float32),
            pltpu.VMEM((CHUNK, DP), jnp.float32),
            pltpu.VMEM((CHUNK, DP), jnp.float32),
            pltpu.VMEM((CHUNK, DP), jnp.float32),
            pltpu.VMEM((TOT,), jnp.float32),
            pltpu.SemaphoreType.DMA,
            pltpu.SemaphoreType.DMA,
        ],
    )
    return k(pos_h, pos_t, pos_r, neg_h, neg_t, neg_r, ent_pad, rel_pad)


def kernel(pos_h, pos_t, pos_r, neg_h, neg_t, neg_r, ent_emb, rel_emb):
    idx = [jnp.asarray(a, jnp.int32)
           for a in (pos_h, pos_t, pos_r, neg_h, neg_t, neg_r)]
    return _run(*idx, ent_emb, rel_emb)
